# baseline (speedup 1.0000x reference)
import jax
import jax.numpy as jnp
from jax.experimental import pallas as pl
from jax.experimental.pallas import tpu as pltpu

_CC, _NN, _FF = 64, 256, 16
_GROUPS = ((0, 0, 1), (1, 1, 12), (2, 12, 16))


def _spatial_conv_kernel(y3_ref, infos_ref, wsd_ref, bd_ref, wa_ref,
                         ba_ref, out_ref):
    C, N = _CC, _NN
    W_sd = wsd_ref[...]
    W_apply = wa_ref[...]
    bd = bd_ref[0, :][:, None]
    ba = ba_ref[0, :][:, None]
    f32 = jnp.float32
    for g, f0, f1 in _GROUPS:
        Az = y3_ref[0, g] != 0
        A = (Az | Az.T).astype(f32)
        deg = jnp.sum(A, axis=1)
        rdeg = (1.0 / jnp.maximum(deg, 1.0))[None, :]
        iso = (deg == 0.0).astype(f32)[None, :]
        for f in range(f0, f1):
            G = infos_ref[0, :, f, :]
            P = jnp.dot(W_sd, G, preferred_element_type=f32)
            M1 = G * (P[:C] + bd)
            Pdst = P[C:]
            t1 = jnp.dot(M1, A, preferred_element_type=f32)
            tG = jnp.dot(G, A, preferred_element_type=f32)
            red = (t1 + tG * Pdst) * rdeg + G * iso
            applied = jnp.dot(W_apply, red, preferred_element_type=f32)
            out_ref[0, :, f, :] = jax.nn.relu(applied + ba)


@jax.jit
def kernel(Y, infos, W_dense, b_dense, W_apply, b_apply):
    B, C, F, N = infos.shape
    W_src = W_dense[:, :C] + W_dense[:, 2 * C:]
    W_dst = W_dense[:, C:2 * C] - W_dense[:, 2 * C:]
    W_sd = jnp.concatenate([W_src, W_dst], axis=0)
    out = pl.pallas_call(
        _spatial_conv_kernel,
        grid=(B,),
        in_specs=[
            pl.BlockSpec((1, 3, N, N), lambda b: (b, 0, 0, 0)),
            pl.BlockSpec((1, C, F, N), lambda b: (b, 0, 0, 0)),
            pl.BlockSpec((2 * C, C), lambda b: (0, 0)),
            pl.BlockSpec((1, C), lambda b: (0, 0)),
            pl.BlockSpec((C, C), lambda b: (0, 0)),
            pl.BlockSpec((1, C), lambda b: (0, 0)),
        ],
        out_specs=pl.BlockSpec((1, C, F, N), lambda b: (b, 0, 0, 0)),
        out_shape=jax.ShapeDtypeStruct((B, C, F, N), jnp.float32),
        compiler_params=pltpu.CompilerParams(
            dimension_semantics=("arbitrary",)),
    )(Y, infos, W_sd, b_dense.reshape(1, C),
      W_apply, b_apply.reshape(1, C))
    return out


# speedup vs baseline: 8.2143x; 8.2143x over previous
import jax
import jax.numpy as jnp
from jax.experimental import pallas as pl
from jax.experimental.pallas import tpu as pltpu

_CC, _NN, _FF = 64, 256, 16
_GROUPS = ((0, 0, 1), (1, 1, 12), (2, 12, 16))


def _spatial_conv_kernel(y3_ref, infos_ref, wsd_ref, bd_ref, wa_ref,
                         ba_ref, out_ref, gflat_ref, oflat_ref):
    C, N, F = _CC, _NN, _FF
    W_sd = wsd_ref[...]
    W_apply = wa_ref[...]
    bd = bd_ref[0, :][:, None]
    ba = ba_ref[0, :][:, None]
    f32 = jnp.float32

    for ct in range(C // 8):
        v = infos_ref[0, ct * 8:(ct + 1) * 8, :, :]
        gflat_ref[ct * 8:(ct + 1) * 8, :] = v.reshape(8, F * N)

    for g, f0, f1 in _GROUPS:
        Az = y3_ref[0, g] != 0
        A = (Az | Az.T).astype(f32)
        deg = jnp.sum(A, axis=1)
        rdeg = (1.0 / jnp.maximum(deg, 1.0))[None, :]
        iso = (deg == 0.0).astype(f32)[None, :]
        for f in range(f0, f1):
            G = gflat_ref[:, pl.ds(f * N, N)]
            P = jnp.dot(W_sd, G, preferred_element_type=f32)
            M1 = G * (P[:C] + bd)
            Pdst = P[C:]
            t1 = jnp.dot(M1, A, preferred_element_type=f32)
            tG = jnp.dot(G, A, preferred_element_type=f32)
            red = (t1 + tG * Pdst) * rdeg + G * iso
            applied = jnp.dot(W_apply, red, preferred_element_type=f32)
            oflat_ref[:, pl.ds(f * N, N)] = jax.nn.relu(applied + ba)

    for ct in range(C // 8):
        w = oflat_ref[ct * 8:(ct + 1) * 8, :]
        out_ref[0, ct * 8:(ct + 1) * 8, :, :] = w.reshape(8, F, N)


@jax.jit
def kernel(Y, infos, W_dense, b_dense, W_apply, b_apply):
    B, C, F, N = infos.shape
    W_src = W_dense[:, :C] + W_dense[:, 2 * C:]
    W_dst = W_dense[:, C:2 * C] - W_dense[:, 2 * C:]
    W_sd = jnp.concatenate([W_src, W_dst], axis=0)
    out = pl.pallas_call(
        _spatial_conv_kernel,
        grid=(B,),
        in_specs=[
            pl.BlockSpec((1, 3, N, N), lambda b: (b, 0, 0, 0)),
            pl.BlockSpec((1, C, F, N), lambda b: (b, 0, 0, 0)),
            pl.BlockSpec((2 * C, C), lambda b: (0, 0)),
            pl.BlockSpec((1, C), lambda b: (0, 0)),
            pl.BlockSpec((C, C), lambda b: (0, 0)),
            pl.BlockSpec((1, C), lambda b: (0, 0)),
        ],
        out_specs=pl.BlockSpec((1, C, F, N), lambda b: (b, 0, 0, 0)),
        out_shape=jax.ShapeDtypeStruct((B, C, F, N), jnp.float32),
        scratch_shapes=[pltpu.VMEM((C, F * N), jnp.float32),
                        pltpu.VMEM((C, F * N), jnp.float32)],
        compiler_params=pltpu.CompilerParams(
            dimension_semantics=("arbitrary",)),
    )(Y, infos, W_sd, b_dense.reshape(1, C),
      W_apply, b_apply.reshape(1, C))
    return out


# speedup vs baseline: 8.9155x; 1.0854x over previous
import jax
import jax.numpy as jnp
from jax.experimental import pallas as pl
from jax.experimental.pallas import tpu as pltpu

_CC, _NN, _FF = 64, 256, 16
_GROUPS = ((0, 0, 1), (1, 1, 12), (2, 12, 16))


def _spatial_conv_kernel(y3_ref, infos_ref, wd_ref, bd_ref, wa_ref,
                         ba_ref, out_ref, gflat_ref, oflat_ref, wsd_ref):
    C, N, F = _CC, _NN, _FF

    @pl.when(pl.program_id(0) == 0)
    def _prep():
        wsd_ref[:C] = wd_ref[:, :C] + wd_ref[:, 2 * C:]
        wsd_ref[C:] = wd_ref[:, C:2 * C] - wd_ref[:, 2 * C:]

    W_sd = wsd_ref[...]
    W_apply = wa_ref[...]
    bd = bd_ref[0, :][:, None]
    ba = ba_ref[0, :][:, None]
    f32 = jnp.float32

    for ct in range(C // 8):
        v = infos_ref[0, ct * 8:(ct + 1) * 8, :, :]
        gflat_ref[ct * 8:(ct + 1) * 8, :] = v.reshape(8, F * N)

    for g, f0, f1 in _GROUPS:
        Az = y3_ref[0, g] != 0
        A = (Az | Az.T).astype(f32)
        deg = jnp.sum(A, axis=1)
        rdeg = (1.0 / jnp.maximum(deg, 1.0))[None, :]
        iso = (deg == 0.0).astype(f32)[None, :]
        for f in range(f0, f1):
            G = gflat_ref[:, pl.ds(f * N, N)]
            P = jnp.dot(W_sd, G, preferred_element_type=f32)
            M1 = G * (P[:C] + bd)
            Pdst = P[C:]
            t1 = jnp.dot(M1, A, preferred_element_type=f32)
            tG = jnp.dot(G, A, preferred_element_type=f32)
            red = (t1 + tG * Pdst) * rdeg + G * iso
            applied = jnp.dot(W_apply, red, preferred_element_type=f32)
            oflat_ref[:, pl.ds(f * N, N)] = jax.nn.relu(applied + ba)

    for ct in range(C // 8):
        w = oflat_ref[ct * 8:(ct + 1) * 8, :]
        out_ref[0, ct * 8:(ct + 1) * 8, :, :] = w.reshape(8, F, N)


@jax.jit
def kernel(Y, infos, W_dense, b_dense, W_apply, b_apply):
    B, C, F, N = infos.shape
    out = pl.pallas_call(
        _spatial_conv_kernel,
        grid=(B,),
        in_specs=[
            pl.BlockSpec((1, 3, N, N), lambda b: (b, 0, 0, 0)),
            pl.BlockSpec((1, C, F, N), lambda b: (b, 0, 0, 0)),
            pl.BlockSpec((C, 3 * C), lambda b: (0, 0)),
            pl.BlockSpec((1, C), lambda b: (0, 0)),
            pl.BlockSpec((C, C), lambda b: (0, 0)),
            pl.BlockSpec((1, C), lambda b: (0, 0)),
        ],
        out_specs=pl.BlockSpec((1, C, F, N), lambda b: (b, 0, 0, 0)),
        out_shape=jax.ShapeDtypeStruct((B, C, F, N), jnp.float32),
        scratch_shapes=[pltpu.VMEM((C, F * N), jnp.float32),
                        pltpu.VMEM((C, F * N), jnp.float32),
                        pltpu.VMEM((2 * C, C), jnp.float32)],
        compiler_params=pltpu.CompilerParams(
            dimension_semantics=("arbitrary",)),
    )(Y, infos, W_dense, b_dense.reshape(1, C),
      W_apply, b_apply.reshape(1, C))
    return out
